# Initial kernel scaffold; baseline (speedup 1.0000x reference)
import functools

import jax
import jax.numpy as jnp
from jax import lax
from jax.experimental import pallas as pl
from jax.experimental.pallas import tpu as pltpu
from jax.experimental.pallas import tpu_sc as plsc

B = 1024
NUM_CLASSES = 1000
CONTEXT_LEN = 77
CTX_DIM = 512
N_CTX = 8
NC, NS = 2, 16
NW = NC * NS
LPW = B // NW
TP_PAD = 80
SUF1 = CONTEXT_LEN - N_CTX


def _body(labels_hbm, table_hbm, tp_hbm, ctx_hbm, out_hbm,
          labels_v, tokens_v, ctx_v, buf, sem):
    wid = lax.axis_index("s") * NC + lax.axis_index("c")
    base = wid * LPW
    pltpu.sync_copy(labels_hbm.at[pl.ds(base, LPW)], labels_v)
    pltpu.async_copy(tp_hbm.at[labels_v], tokens_v, sem).wait()
    pltpu.sync_copy(ctx_hbm, ctx_v)
    pltpu.sync_copy(ctx_hbm, buf.at[pl.ds(1, N_CTX)])

    def body(i, carry):
        pltpu.async_copy(
            table_hbm.at[tokens_v.at[i, pl.ds(N_CTX, SUF1)]],
            buf.at[pl.ds(N_CTX, SUF1)], sem).wait()
        pltpu.sync_copy(ctx_v.at[pl.ds(N_CTX - 1, 1)],
                        buf.at[pl.ds(N_CTX, 1)])
        pltpu.async_copy(
            table_hbm.at[tokens_v.at[i, pl.ds(0, 1)]],
            buf.at[pl.ds(0, 1)], sem).wait()
        pltpu.sync_copy(buf, out_hbm.at[base + i])
        return carry

    lax.fori_loop(0, LPW, body, 0)


def kernel(labels, token_embedding, tokenized_prompts, ctx):
    tp_pad = jnp.pad(tokenized_prompts, ((0, 0), (0, TP_PAD - CONTEXT_LEN)))
    mesh = plsc.VectorSubcoreMesh(core_axis_name="c", subcore_axis_name="s")
    k = functools.partial(
        pl.kernel,
        out_type=jax.ShapeDtypeStruct((B, CONTEXT_LEN, CTX_DIM), jnp.float32),
        mesh=mesh,
        scratch_types=[
            pltpu.VMEM((LPW,), jnp.int32),
            pltpu.VMEM((LPW, TP_PAD), jnp.int32),
            pltpu.VMEM((N_CTX, CTX_DIM), jnp.float32),
            pltpu.VMEM((CONTEXT_LEN, CTX_DIM), jnp.float32),
            pltpu.SemaphoreType.DMA,
        ],
    )(_body)
    return k(labels, token_embedding, tp_pad, ctx)


# speedup vs baseline: 1.0307x; 1.0307x over previous
import functools

import jax
import jax.numpy as jnp
from jax import lax
from jax.experimental import pallas as pl
from jax.experimental.pallas import tpu as pltpu
from jax.experimental.pallas import tpu_sc as plsc

B = 1024
NUM_CLASSES = 1000
CONTEXT_LEN = 77
CTX_DIM = 512
N_CTX = 8
NC, NS = 2, 16
NW = NC * NS
LPW = B // NW
TP_PAD = 80
SUF = TP_PAD - N_CTX


def _body(labels_hbm, table_hbm, tp_hbm, ctx_hbm, out_hbm,
          labels_v, tokens_v, ctx_v, buf, sem):
    wid = lax.axis_index("s") * NC + lax.axis_index("c")
    base = wid * LPW
    pltpu.sync_copy(labels_hbm.at[pl.ds(base, LPW)], labels_v)
    pltpu.async_copy(tp_hbm.at[labels_v], tokens_v, sem).wait()
    pltpu.sync_copy(ctx_hbm, ctx_v)

    def body(i, carry):
        b = base + i
        pltpu.async_copy(
            table_hbm.at[tokens_v.at[i, pl.ds(N_CTX, SUF)]],
            buf.at[pl.ds(0, SUF)], sem).wait()
        pltpu.async_copy(
            table_hbm.at[tokens_v.at[i, pl.ds(0, N_CTX)]],
            buf.at[pl.ds(SUF, N_CTX)], sem).wait()
        pltpu.sync_copy(buf.at[pl.ds(SUF, 1)], out_hbm.at[b, pl.ds(0, 1)])
        pltpu.sync_copy(ctx_v, out_hbm.at[b, pl.ds(1, N_CTX)])
        pltpu.sync_copy(buf.at[pl.ds(1, SUF - 4)],
                        out_hbm.at[b, pl.ds(N_CTX + 1, SUF - 4)])
        return carry

    lax.fori_loop(0, LPW, body, 0)


def kernel(labels, token_embedding, tokenized_prompts, ctx):
    tp_pad = jnp.pad(tokenized_prompts, ((0, 0), (0, TP_PAD - CONTEXT_LEN)))
    mesh = plsc.VectorSubcoreMesh(core_axis_name="c", subcore_axis_name="s")
    k = functools.partial(
        pl.kernel,
        out_type=jax.ShapeDtypeStruct((B, CONTEXT_LEN, CTX_DIM), jnp.float32),
        mesh=mesh,
        scratch_types=[
            pltpu.VMEM((LPW,), jnp.int32),
            pltpu.VMEM((LPW, TP_PAD), jnp.int32),
            pltpu.VMEM((N_CTX, CTX_DIM), jnp.float32),
            pltpu.VMEM((TP_PAD, CTX_DIM), jnp.float32),
            pltpu.SemaphoreType.DMA,
        ],
        compiler_params=pltpu.CompilerParams(use_tc_tiling_on_sc=False),
    )(_body)
    return k(labels, token_embedding, tp_pad, ctx)


# speedup vs baseline: 1.0357x; 1.0048x over previous
import functools

import jax
import jax.numpy as jnp
from jax import lax
from jax.experimental import pallas as pl
from jax.experimental.pallas import tpu as pltpu
from jax.experimental.pallas import tpu_sc as plsc

B = 1024
CONTEXT_LEN = 77
CTX_DIM = 512
N_CTX = 8
NC, NS = 2, 16
NW = NC * NS
LPW = B // NW
SUF = CONTEXT_LEN - N_CTX - 1
GW = SUF + 4


def _body(labels_hbm, table_hbm, tp_hbm, ctx_hbm, out_hbm,
          labels_v, tokens_v, ctx_v, s0, s1, gsem0, gsem1, wsem0, wsem1):
    stage = (s0, s1)
    gsem = (gsem0, gsem1)
    wsem = (wsem0, wsem1)
    wid = lax.axis_index("s") * NC + lax.axis_index("c")
    base = wid * LPW
    pltpu.sync_copy(labels_hbm.at[pl.ds(base, LPW)], labels_v)
    pltpu.async_copy(tp_hbm.at[labels_v], tokens_v, gsem0).wait()
    pltpu.sync_copy(ctx_hbm, ctx_v)

    def issue_writes(s, b, sem):
        pltpu.async_copy(s.at[pl.ds(0, 1)], out_hbm.at[b, pl.ds(0, 1)], sem)
        pltpu.async_copy(ctx_v, out_hbm.at[b, pl.ds(1, N_CTX)], sem)
        pltpu.async_copy(s.at[pl.ds(1, SUF)],
                         out_hbm.at[b, pl.ds(1 + N_CTX, SUF)], sem)

    def drain_writes(s, b, sem):
        pltpu.make_async_copy(s.at[pl.ds(0, 1)],
                              out_hbm.at[b, pl.ds(0, 1)], sem).wait()
        pltpu.make_async_copy(ctx_v, out_hbm.at[b, pl.ds(1, N_CTX)],
                              sem).wait()
        pltpu.make_async_copy(s.at[pl.ds(1, SUF)],
                              out_hbm.at[b, pl.ds(1 + N_CTX, SUF)],
                              sem).wait()

    def body(g, carry):
        for p in range(2):
            i = 2 * g + p
            b = base + i

            @pl.when(g > 0)
            def _():
                drain_writes(stage[p], b - 2, wsem[p])

            pltpu.async_copy(table_hbm.at[tokens_v.at[i]],
                             stage[p], gsem[p]).wait()
            issue_writes(stage[p], b, wsem[p])
        return carry

    lax.fori_loop(0, LPW // 2, body, 0)
    for p in range(2):
        drain_writes(stage[p], base + LPW - 2 + p, wsem[p])


def kernel(labels, token_embedding, tokenized_prompts, ctx):
    tp_perm = jnp.concatenate(
        [tokenized_prompts[:, :1],
         tokenized_prompts[:, 1 + N_CTX:],
         jnp.zeros((tokenized_prompts.shape[0], 3), jnp.int32)], axis=1)
    mesh = plsc.VectorSubcoreMesh(core_axis_name="c", subcore_axis_name="s")
    k = functools.partial(
        pl.kernel,
        out_type=jax.ShapeDtypeStruct((B, CONTEXT_LEN, CTX_DIM), jnp.float32),
        mesh=mesh,
        scratch_types=[
            pltpu.VMEM((LPW,), jnp.int32),
            pltpu.VMEM((LPW, GW), jnp.int32),
            pltpu.VMEM((N_CTX, CTX_DIM), jnp.float32),
            pltpu.VMEM((GW, CTX_DIM), jnp.float32),
            pltpu.VMEM((GW, CTX_DIM), jnp.float32),
            pltpu.SemaphoreType.DMA,
            pltpu.SemaphoreType.DMA,
            pltpu.SemaphoreType.DMA,
            pltpu.SemaphoreType.DMA,
        ],
        compiler_params=pltpu.CompilerParams(use_tc_tiling_on_sc=False),
    )(_body)
    return k(labels, token_embedding, tp_perm, ctx)
